# baseline (speedup 1.0000x reference)
import jax
import jax.numpy as jnp
from jax.experimental import pallas as pl
from jax.experimental.pallas import tpu as pltpu

GENOMES = 16
FEATURES = 128
EMBED = 16
BATCH = 16384

BT = 8192


def _embed_kernel(x_ref, w_ref, o_ref):
    o_ref[0] = jnp.dot(x_ref[0], w_ref[0], preferred_element_type=jnp.float32)


@jax.jit
def kernel(tensor, W):
    grid = (GENOMES, BATCH // BT)
    return pl.pallas_call(
        _embed_kernel,
        grid=grid,
        in_specs=[
            pl.BlockSpec((1, BT, FEATURES), lambda g, b: (g, b, 0)),
            pl.BlockSpec((1, FEATURES, EMBED), lambda g, b: (g, 0, 0)),
        ],
        out_specs=pl.BlockSpec((1, BT, EMBED), lambda g, b: (g, b, 0)),
        out_shape=jax.ShapeDtypeStruct((GENOMES, BATCH, EMBED), jnp.float32),
        compiler_params=pltpu.CompilerParams(
            dimension_semantics=(pltpu.PARALLEL, pltpu.PARALLEL),
        ),
    )(tensor, W)


# speedup vs baseline: 3.1996x; 3.1996x over previous
import jax
import jax.numpy as jnp
from jax import lax
from jax.experimental import pallas as pl
from jax.experimental.pallas import tpu as pltpu

GENOMES = 16
FEATURES = 128
EMBED = 16
BATCH = 16384

BT = 2048
PER_G = BATCH // BT
STEPS = GENOMES * PER_G
NBUF = 8
OB = 4


def _embed_kernel(x_hbm, w_ref, o_hbm, *scratch):
    xbufs = scratch[:NBUF]
    obufs = scratch[NBUF:NBUF + OB]
    in_sems = scratch[NBUF + OB:2 * NBUF + OB]
    out_sems = scratch[2 * NBUF + OB:]

    def in_copy(s):
        g, r = divmod(s, PER_G)
        return pltpu.make_async_copy(
            x_hbm.at[g, pl.ds(r * BT, BT), :], xbufs[s % NBUF],
            in_sems[s % NBUF])

    def out_copy(s):
        g, r = divmod(s, PER_G)
        return pltpu.make_async_copy(
            obufs[s % OB], o_hbm.at[g, :, pl.ds(r * BT, BT)],
            out_sems[s % OB])

    for s in range(NBUF):
        in_copy(s).start()

    for s in range(STEPS):
        g = s // PER_G
        if s >= OB:
            out_copy(s - OB).wait()
        in_copy(s).wait()
        obufs[s % OB][...] = lax.dot_general(
            w_ref[g], xbufs[s % NBUF][...],
            dimension_numbers=(((0,), (1,)), ((), ())),
            preferred_element_type=jnp.float32)
        out_copy(s).start()
        if s + NBUF < STEPS:
            in_copy(s + NBUF).start()

    for s in range(STEPS - OB, STEPS):
        out_copy(s).wait()


@jax.jit
def kernel(tensor, W):
    scratch = (
        [pltpu.VMEM((BT, FEATURES), jnp.float32)] * NBUF
        + [pltpu.VMEM((EMBED, BT), jnp.float32)] * OB
        + [pltpu.SemaphoreType.DMA] * (NBUF + OB)
    )
    out_t = pl.pallas_call(
        _embed_kernel,
        in_specs=[
            pl.BlockSpec(memory_space=pl.ANY),
            pl.BlockSpec(memory_space=pltpu.VMEM),
        ],
        out_specs=pl.BlockSpec(memory_space=pl.ANY),
        out_shape=jax.ShapeDtypeStruct((GENOMES, EMBED, BATCH), jnp.float32),
        scratch_shapes=scratch,
    )(tensor, W)
    return out_t.transpose(0, 2, 1)
